# baseline (speedup 1.0000x reference)
import functools

import jax
import jax.numpy as jnp
from jax import lax
from jax.experimental import pallas as pl
from jax.experimental.pallas import tpu as pltpu
from jax.experimental.pallas import tpu_sc as plsc

EPS = 1e-5
G = 64
LANES = 16
CHUNK = 128
NBUF = 2
NSLOT = 4
NTILES = 32
TPC = 16
WCHUNK = 128


def _round_up(a, b):
    return (a + b - 1) // b * b


def _deg_body(dst_hbm, deg_hbm, dst_v, deg_v):
    cid = lax.axis_index("c")
    sid = lax.axis_index("s")
    wid = cid * TPC + sid
    npad = deg_v.shape[0]
    ept = dst_v.shape[0]

    def zero(i, carry):
        deg_v[pl.ds(i * LANES, LANES)] = jnp.zeros((LANES,), jnp.float32)
        return carry

    lax.fori_loop(0, npad // LANES, zero, 0)
    pltpu.sync_copy(dst_hbm.at[wid], dst_v)

    ones = jnp.ones((LANES,), jnp.float32)

    def accum(i, carry):
        idx = dst_v[pl.ds(i * LANES, LANES)]
        plsc.addupdate_scatter(deg_v, [idx], ones)
        return carry

    lax.fori_loop(0, ept // LANES, accum, 0)
    pltpu.sync_copy(deg_v, deg_hbm.at[wid])


def _spmm_body(src_hbm, dst_hbm, table_hbm, out_hbm, sidx_v, didx_v, rows_v,
               acc_sh, gsem, isem):
    cid = lax.axis_index("c")
    sid = lax.axis_index("s")
    wid = cid * TPC + sid
    k = src_hbm.shape[1]
    ng = k // NBUF
    npad = acc_sh.shape[0]
    rows_per_tile = npad // TPC

    def idx_issue(g, sl):
        gr = pl.ds(g * NBUF, NBUF)
        pltpu.async_copy(src_hbm.at[wid, gr], sidx_v.at[sl], isem.at[sl])
        pltpu.async_copy(dst_hbm.at[wid, gr], didx_v.at[sl], isem.at[sl])

    def idx_wait(sl):
        pltpu.make_async_copy(src_hbm.at[wid, pl.ds(0, NBUF)], sidx_v.at[sl],
                              isem.at[sl]).wait()
        pltpu.make_async_copy(dst_hbm.at[wid, pl.ds(0, NBUF)], didx_v.at[sl],
                              isem.at[sl]).wait()

    def zero(r, carry):
        for c in range(rows_v.shape[2] // LANES):
            rows_v[0, r, pl.ds(c * LANES, LANES)] = jnp.zeros((LANES,),
                                                              jnp.float32)
        return carry

    lax.fori_loop(0, CHUNK, zero, 0)
    base = sid * rows_per_tile
    for i in range(rows_per_tile // CHUNK):
        pltpu.sync_copy(rows_v.at[0], acc_sh.at[pl.ds(base + i * CHUNK,
                                                      CHUNK)])

    idx_issue(0, 0)
    idx_issue(1, 1)
    plsc.subcore_barrier()

    idx_wait(0)
    for b in range(NBUF):
        pltpu.async_copy(table_hbm.at[sidx_v.at[0, b]], rows_v.at[b],
                         gsem.at[b])

    def group(g, carry):
        csl = jnp.bitwise_and(g, NSLOT - 1)
        nsl = jnp.bitwise_and(g + 1, NSLOT - 1)

        @pl.when(g + 2 < ng)
        def _():
            idx_issue(g + 2, jnp.bitwise_and(g + 2, NSLOT - 1))

        @pl.when(g + 1 < ng)
        def _():
            idx_wait(nsl)

        for b in range(NBUF):
            pltpu.make_async_copy(table_hbm.at[sidx_v.at[csl, b]],
                                  rows_v.at[b], gsem.at[b]).wait()
            pltpu.sync_copy(rows_v.at[b], acc_sh.at[didx_v.at[csl, b]],
                            add=True)

            @pl.when(g + 1 < ng)
            def _():
                pltpu.async_copy(table_hbm.at[sidx_v.at[nsl, b]],
                                 rows_v.at[b], gsem.at[b])
        return carry

    lax.fori_loop(0, ng, group, 0)
    plsc.subcore_barrier()

    for i in range(rows_per_tile // WCHUNK):
        sl = pl.ds(base + i * WCHUNK, WCHUNK)
        pltpu.sync_copy(acc_sh.at[sl], out_hbm.at[cid, sl])


def _tc1_body(x_ref, w_ref, b_ref, degp_ref, xs_ref, dinvb_ref):
    deg = jnp.sum(degp_ref[...], axis=0) + 1.0
    dinv = lax.rsqrt(jnp.maximum(deg, 1.0))
    dinv_b = jnp.broadcast_to(dinv[:, None], xs_ref.shape)
    xw = jnp.dot(x_ref[...], w_ref[...],
                 preferred_element_type=jnp.float32) + b_ref[...]
    dinvb_ref[...] = dinv_b
    xs_ref[...] = xw * dinv_b


def _bn_relu(agg, g_ref, be_ref, m_ref, v_ref):
    s = g_ref[...] * lax.rsqrt(v_ref[...] + EPS)
    t = be_ref[...] - m_ref[...] * s
    return jnp.maximum(agg * s + t, 0.0)


def _tc_mid_body(acc_ref, xs_ref, dinvb_ref, g_ref, be_ref, m_ref, v_ref,
                 w_ref, b_ref, out_ref):
    agg = (acc_ref[0] + acc_ref[1] + xs_ref[...]) * dinvb_ref[...]
    h = _bn_relu(agg, g_ref, be_ref, m_ref, v_ref)
    xw = jnp.dot(h, w_ref[...], preferred_element_type=jnp.float32) + b_ref[...]
    out_ref[...] = xw * dinvb_ref[...]


def _tc_final_body(acc_ref, xs_ref, dinvb_ref, g_ref, be_ref, m_ref, v_ref,
                   batch_ref, out_ref):
    agg = (acc_ref[0] + acc_ref[1] + xs_ref[...]) * dinvb_ref[...]
    h = _bn_relu(agg, g_ref, be_ref, m_ref, v_ref)
    gids = lax.broadcasted_iota(jnp.int32, (G, batch_ref.shape[1]),
                                0).astype(jnp.float32)
    p = jnp.where(batch_ref[...] == gids, 1.0, 0.0)
    sums = jnp.dot(p, h, preferred_element_type=jnp.float32)
    cnt = jnp.sum(p, axis=1, keepdims=True)
    out_ref[...] = sums / jnp.maximum(cnt, 1.0)


def _tc_call(body, out_shape):
    return pl.pallas_call(body, out_shape=out_shape)


def kernel(x, edge_index, batch,
           W1, b1, gamma1, beta1, mean1, var1,
           W2, b2, gamma2, beta2, mean2, var2,
           W3, b3, gamma3, beta3, mean3, var3):
    n, d = x.shape
    h = W1.shape[1]
    e = edge_index.shape[1]

    npad = _round_up(n + 1, TPC * WCHUNK)
    ep = _round_up(e, NTILES * NBUF * CHUNK)
    ept = ep // NTILES
    k = ept // CHUNK

    pad_e = ep - e
    pad_idx = n + jnp.arange(pad_e, dtype=jnp.int32) % (npad - n)
    src_p = jnp.concatenate([edge_index[0], pad_idx])
    dst_p = jnp.concatenate([edge_index[1], pad_idx])
    src3 = src_p.reshape(NTILES, k, CHUNK)
    dst3 = dst_p.reshape(NTILES, k, CHUNK)
    dstf = dst_p.reshape(NTILES, ept)
    x_p = jnp.zeros((npad, d), jnp.float32).at[:n].set(x)
    batch_row = jnp.concatenate(
        [batch, jnp.full((npad - n,), G, jnp.int32)]).astype(
            jnp.float32).reshape(1, npad)
    row = lambda v: v.reshape(1, h)

    mesh = plsc.VectorSubcoreMesh(core_axis_name="c", subcore_axis_name="s")

    deg_call = pl.kernel(
        _deg_body,
        out_type=jax.ShapeDtypeStruct((NTILES, npad), jnp.float32),
        mesh=mesh,
        compiler_params=pltpu.CompilerParams(needs_layout_passes=False),
        scratch_types=[
            pltpu.VMEM((ept,), jnp.int32),
            pltpu.VMEM((npad,), jnp.float32),
        ],
    )

    spmm_call = pl.kernel(
        _spmm_body,
        out_type=jax.ShapeDtypeStruct((2, npad, h), jnp.float32),
        mesh=mesh,
        scratch_types=[
            pltpu.VMEM((NSLOT, NBUF, CHUNK), jnp.int32),
            pltpu.VMEM((NSLOT, NBUF, CHUNK), jnp.int32),
            pltpu.VMEM((NBUF, CHUNK, h), jnp.float32),
            pltpu.VMEM_SHARED((npad, h), jnp.float32),
            pltpu.SemaphoreType.DMA((NBUF,)),
            pltpu.SemaphoreType.DMA((NSLOT,)),
        ],
    )

    deg_parts = deg_call(dstf)

    xs1, dinvb = _tc_call(
        _tc1_body,
        (jax.ShapeDtypeStruct((npad, h), jnp.float32),
         jax.ShapeDtypeStruct((npad, h), jnp.float32)),
    )(x_p, W1, row(b1), deg_parts)

    acc1 = spmm_call(src3, dst3, xs1)
    xs2 = _tc_call(_tc_mid_body, jax.ShapeDtypeStruct((npad, h), jnp.float32))(
        acc1, xs1, dinvb, row(gamma1), row(beta1), row(mean1), row(var1),
        W2, row(b2))

    acc2 = spmm_call(src3, dst3, xs2)
    xs3 = _tc_call(_tc_mid_body, jax.ShapeDtypeStruct((npad, h), jnp.float32))(
        acc2, xs2, dinvb, row(gamma2), row(beta2), row(mean2), row(var2),
        W3, row(b3))

    acc3 = spmm_call(src3, dst3, xs3)
    out = _tc_call(_tc_final_body, jax.ShapeDtypeStruct((G, h), jnp.float32))(
        acc3, xs3, dinvb, row(gamma3), row(beta3), row(mean3), row(var3),
        batch_row)
    return out


# speedup vs baseline: 1.1177x; 1.1177x over previous
import functools

import jax
import jax.numpy as jnp
from jax import lax
from jax.experimental import pallas as pl
from jax.experimental.pallas import tpu as pltpu
from jax.experimental.pallas import tpu_sc as plsc

EPS = 1e-5
G = 64
LANES = 16
CHUNK = 64
NBUF = 4
NSLOT = 4
NTILES = 32
TPC = 16
WCHUNK = 128


def _round_up(a, b):
    return (a + b - 1) // b * b


def _deg_body(dst_hbm, deg_hbm, dst_v, deg_v):
    cid = lax.axis_index("c")
    sid = lax.axis_index("s")
    wid = cid * TPC + sid
    npad = deg_v.shape[0]
    ept = dst_v.shape[0]

    def zero(i, carry):
        deg_v[pl.ds(i * LANES, LANES)] = jnp.zeros((LANES,), jnp.float32)
        return carry

    lax.fori_loop(0, npad // LANES, zero, 0)
    pltpu.sync_copy(dst_hbm.at[wid], dst_v)

    ones = jnp.ones((LANES,), jnp.float32)

    def accum(i, carry):
        idx = dst_v[pl.ds(i * LANES, LANES)]
        plsc.addupdate_scatter(deg_v, [idx], ones)
        return carry

    lax.fori_loop(0, ept // LANES, accum, 0)
    pltpu.sync_copy(deg_v, deg_hbm.at[wid])


def _spmm_body(src_hbm, dst_hbm, table_hbm, out_hbm, sidx_v, didx_v, rows_v,
               acc_sh, gsem, isem):
    cid = lax.axis_index("c")
    sid = lax.axis_index("s")
    wid = cid * TPC + sid
    k = src_hbm.shape[1]
    ng = k // NBUF
    npad = acc_sh.shape[0]
    rows_per_tile = npad // TPC

    def idx_issue(g, sl):
        gr = pl.ds(g * NBUF, NBUF)
        pltpu.async_copy(src_hbm.at[wid, gr], sidx_v.at[sl], isem.at[sl])
        pltpu.async_copy(dst_hbm.at[wid, gr], didx_v.at[sl], isem.at[sl])

    def idx_wait(sl):
        pltpu.make_async_copy(src_hbm.at[wid, pl.ds(0, NBUF)], sidx_v.at[sl],
                              isem.at[sl]).wait()
        pltpu.make_async_copy(dst_hbm.at[wid, pl.ds(0, NBUF)], didx_v.at[sl],
                              isem.at[sl]).wait()

    def zero(r, carry):
        for c in range(rows_v.shape[2] // LANES):
            rows_v[0, r, pl.ds(c * LANES, LANES)] = jnp.zeros((LANES,),
                                                              jnp.float32)
        return carry

    lax.fori_loop(0, CHUNK, zero, 0)
    base = sid * rows_per_tile
    for i in range(rows_per_tile // CHUNK):
        pltpu.sync_copy(rows_v.at[0], acc_sh.at[pl.ds(base + i * CHUNK,
                                                      CHUNK)])

    idx_issue(0, 0)
    idx_issue(1, 1)
    plsc.subcore_barrier()

    idx_wait(0)
    for b in range(NBUF):
        pltpu.async_copy(table_hbm.at[sidx_v.at[0, b]], rows_v.at[b],
                         gsem.at[b])

    def group(g, carry):
        csl = jnp.bitwise_and(g, NSLOT - 1)
        nsl = jnp.bitwise_and(g + 1, NSLOT - 1)

        @pl.when(g + 2 < ng)
        def _():
            idx_issue(g + 2, jnp.bitwise_and(g + 2, NSLOT - 1))

        @pl.when(g + 1 < ng)
        def _():
            idx_wait(nsl)

        for b in range(NBUF):
            pltpu.make_async_copy(table_hbm.at[sidx_v.at[csl, b]],
                                  rows_v.at[b], gsem.at[b]).wait()
            pltpu.sync_copy(rows_v.at[b], acc_sh.at[didx_v.at[csl, b]],
                            add=True)

            @pl.when(g + 1 < ng)
            def _():
                pltpu.async_copy(table_hbm.at[sidx_v.at[nsl, b]],
                                 rows_v.at[b], gsem.at[b])
        return carry

    lax.fori_loop(0, ng, group, 0)
    plsc.subcore_barrier()

    for i in range(rows_per_tile // WCHUNK):
        sl = pl.ds(base + i * WCHUNK, WCHUNK)
        pltpu.sync_copy(acc_sh.at[sl], out_hbm.at[cid, sl])


def _tc1_body(x_ref, w_ref, b_ref, degp_ref, xs_ref, dinvb_ref):
    deg = jnp.sum(degp_ref[...], axis=0) + 1.0
    dinv = lax.rsqrt(jnp.maximum(deg, 1.0))
    dinv_b = jnp.broadcast_to(dinv[:, None], xs_ref.shape)
    xw = jnp.dot(x_ref[...], w_ref[...],
                 preferred_element_type=jnp.float32) + b_ref[...]
    dinvb_ref[...] = dinv_b
    xs_ref[...] = xw * dinv_b


def _bn_relu(agg, g_ref, be_ref, m_ref, v_ref):
    s = g_ref[...] * lax.rsqrt(v_ref[...] + EPS)
    t = be_ref[...] - m_ref[...] * s
    return jnp.maximum(agg * s + t, 0.0)


def _tc_mid_body(acc_ref, xs_ref, dinvb_ref, g_ref, be_ref, m_ref, v_ref,
                 w_ref, b_ref, out_ref):
    agg = (acc_ref[0] + acc_ref[1] + xs_ref[...]) * dinvb_ref[...]
    h = _bn_relu(agg, g_ref, be_ref, m_ref, v_ref)
    xw = jnp.dot(h, w_ref[...], preferred_element_type=jnp.float32) + b_ref[...]
    out_ref[...] = xw * dinvb_ref[...]


def _tc_final_body(acc_ref, xs_ref, dinvb_ref, g_ref, be_ref, m_ref, v_ref,
                   batch_ref, out_ref):
    agg = (acc_ref[0] + acc_ref[1] + xs_ref[...]) * dinvb_ref[...]
    h = _bn_relu(agg, g_ref, be_ref, m_ref, v_ref)
    gids = lax.broadcasted_iota(jnp.int32, (G, batch_ref.shape[1]),
                                0).astype(jnp.float32)
    p = jnp.where(batch_ref[...] == gids, 1.0, 0.0)
    sums = jnp.dot(p, h, preferred_element_type=jnp.float32)
    cnt = jnp.sum(p, axis=1, keepdims=True)
    out_ref[...] = sums / jnp.maximum(cnt, 1.0)


def _tc_call(body, out_shape):
    return pl.pallas_call(body, out_shape=out_shape)


def kernel(x, edge_index, batch,
           W1, b1, gamma1, beta1, mean1, var1,
           W2, b2, gamma2, beta2, mean2, var2,
           W3, b3, gamma3, beta3, mean3, var3):
    n, d = x.shape
    h = W1.shape[1]
    e = edge_index.shape[1]

    npad = _round_up(n + 1, TPC * WCHUNK)
    ep = _round_up(e, NTILES * NBUF * CHUNK)
    ept = ep // NTILES
    k = ept // CHUNK

    pad_e = ep - e
    pad_idx = n + jnp.arange(pad_e, dtype=jnp.int32) % (npad - n)
    src_p = jnp.concatenate([edge_index[0], pad_idx])
    dst_p = jnp.concatenate([edge_index[1], pad_idx])
    src3 = src_p.reshape(NTILES, k, CHUNK)
    dst3 = dst_p.reshape(NTILES, k, CHUNK)
    dstf = edge_index[1].reshape(NTILES, e // NTILES)
    x_p = jnp.zeros((npad, d), jnp.float32).at[:n].set(x)
    batch_row = jnp.concatenate(
        [batch, jnp.full((npad - n,), G, jnp.int32)]).astype(
            jnp.float32).reshape(1, npad)
    row = lambda v: v.reshape(1, h)

    mesh = plsc.VectorSubcoreMesh(core_axis_name="c", subcore_axis_name="s")

    deg_call = pl.kernel(
        _deg_body,
        out_type=jax.ShapeDtypeStruct((NTILES, npad), jnp.float32),
        mesh=mesh,
        compiler_params=pltpu.CompilerParams(needs_layout_passes=False),
        scratch_types=[
            pltpu.VMEM((e // NTILES,), jnp.int32),
            pltpu.VMEM((npad,), jnp.float32),
        ],
    )

    spmm_call = pl.kernel(
        _spmm_body,
        out_type=jax.ShapeDtypeStruct((2, npad, h), jnp.float32),
        mesh=mesh,
        scratch_types=[
            pltpu.VMEM((NSLOT, NBUF, CHUNK), jnp.int32),
            pltpu.VMEM((NSLOT, NBUF, CHUNK), jnp.int32),
            pltpu.VMEM((NBUF, CHUNK, h), jnp.float32),
            pltpu.VMEM_SHARED((npad, h), jnp.float32),
            pltpu.SemaphoreType.DMA((NBUF,)),
            pltpu.SemaphoreType.DMA((NSLOT,)),
        ],
    )

    deg_parts = deg_call(dstf)

    xs1, dinvb = _tc_call(
        _tc1_body,
        (jax.ShapeDtypeStruct((npad, h), jnp.float32),
         jax.ShapeDtypeStruct((npad, h), jnp.float32)),
    )(x_p, W1, row(b1), deg_parts)

    acc1 = spmm_call(src3, dst3, xs1)
    xs2 = _tc_call(_tc_mid_body, jax.ShapeDtypeStruct((npad, h), jnp.float32))(
        acc1, xs1, dinvb, row(gamma1), row(beta1), row(mean1), row(var1),
        W2, row(b2))

    acc2 = spmm_call(src3, dst3, xs2)
    xs3 = _tc_call(_tc_mid_body, jax.ShapeDtypeStruct((npad, h), jnp.float32))(
        acc2, xs2, dinvb, row(gamma2), row(beta2), row(mean2), row(var2),
        W3, row(b3))

    acc3 = spmm_call(src3, dst3, xs3)
    out = _tc_call(_tc_final_body, jax.ShapeDtypeStruct((G, h), jnp.float32))(
        acc3, xs3, dinvb, row(gamma3), row(beta3), row(mean3), row(var3),
        batch_row)
    return out


# speedup vs baseline: 1.1177x; 1.0000x over previous
import jax
import jax.numpy as jnp
from jax import lax
from jax.experimental import pallas as pl
from jax.experimental.pallas import tpu as pltpu
from jax.experimental.pallas import tpu_sc as plsc

EPS = 1e-5
G = 64
LANES = 16
CHUNK = 64
NBUF = 4
NSLOT = 4
NTILES = 32
TPC = 16
WCHUNK = 128


def _round_up(a, b):
    return (a + b - 1) // b * b


def _deg_body(dst_hbm, deg_hbm, dst_v, deg_v):
    cid = lax.axis_index("c")
    sid = lax.axis_index("s")
    wid = cid * TPC + sid
    npad = deg_v.shape[0]
    ept = dst_v.shape[0]

    def zero(i, carry):
        deg_v[pl.ds(i * LANES, LANES)] = jnp.zeros((LANES,), jnp.float32)
        return carry

    lax.fori_loop(0, npad // LANES, zero, 0)
    pltpu.sync_copy(dst_hbm.at[wid], dst_v)

    ones = jnp.ones((LANES,), jnp.float32)

    def accum(i, carry):
        idx = dst_v[pl.ds(i * LANES, LANES)]
        plsc.addupdate_scatter(deg_v, [idx], ones)
        return carry

    lax.fori_loop(0, ept // LANES, accum, 0)
    pltpu.sync_copy(deg_v, deg_hbm.at[wid])


def _spmm_body(src_hbm, dst_hbm, table_hbm, out_hbm, sidx_v, didx_v, rows_v,
               acc_sh, gsem, isem):
    cid = lax.axis_index("c")
    sid = lax.axis_index("s")
    wid = cid * TPC + sid
    k = src_hbm.shape[1]
    ng = k // NBUF
    npad = acc_sh.shape[0]
    rows_per_tile = npad // TPC

    def idx_issue(g, sl):
        gr = pl.ds(g * NBUF, NBUF)
        pltpu.async_copy(src_hbm.at[wid, gr], sidx_v.at[sl], isem.at[sl])
        pltpu.async_copy(dst_hbm.at[wid, gr], didx_v.at[sl], isem.at[sl])

    def idx_wait(sl):
        pltpu.make_async_copy(src_hbm.at[wid, pl.ds(0, NBUF)], sidx_v.at[sl],
                              isem.at[sl]).wait()
        pltpu.make_async_copy(dst_hbm.at[wid, pl.ds(0, NBUF)], didx_v.at[sl],
                              isem.at[sl]).wait()

    def zero(r, carry):
        for c in range(rows_v.shape[2] // LANES):
            rows_v[0, r, pl.ds(c * LANES, LANES)] = jnp.zeros((LANES,),
                                                              jnp.float32)
        return carry

    lax.fori_loop(0, CHUNK, zero, 0)
    base = sid * rows_per_tile
    for i in range(rows_per_tile // CHUNK):
        pltpu.sync_copy(rows_v.at[0], acc_sh.at[pl.ds(base + i * CHUNK,
                                                      CHUNK)])

    idx_issue(0, 0)
    idx_issue(1, 1)
    plsc.subcore_barrier()

    idx_wait(0)
    for b in range(NBUF):
        pltpu.async_copy(table_hbm.at[sidx_v.at[0, b]], rows_v.at[b],
                         gsem.at[b])

    def group(g, carry):
        csl = jnp.bitwise_and(g, NSLOT - 1)
        nsl = jnp.bitwise_and(g + 1, NSLOT - 1)

        @pl.when(g + 2 < ng)
        def _():
            idx_issue(g + 2, jnp.bitwise_and(g + 2, NSLOT - 1))

        @pl.when(g + 1 < ng)
        def _():
            idx_wait(nsl)

        for b in range(NBUF):
            pltpu.make_async_copy(table_hbm.at[sidx_v.at[csl, b]],
                                  rows_v.at[b], gsem.at[b]).wait()
            pltpu.sync_copy(rows_v.at[b], acc_sh.at[didx_v.at[csl, b]],
                            add=True)

            @pl.when(g + 1 < ng)
            def _():
                pltpu.async_copy(table_hbm.at[sidx_v.at[nsl, b]],
                                 rows_v.at[b], gsem.at[b])
        return carry

    lax.fori_loop(0, ng, group, 0)
    plsc.subcore_barrier()

    for i in range(rows_per_tile // WCHUNK):
        sl = pl.ds(base + i * WCHUNK, WCHUNK)
        pltpu.sync_copy(acc_sh.at[sl], out_hbm.at[cid, sl])


def _tc1_body(x_ref, w_ref, b_ref, degp_ref, xs_ref, dinvb_ref):
    deg = jnp.sum(degp_ref[...], axis=0) + 1.0
    dinv = lax.rsqrt(jnp.maximum(deg, 1.0))
    dinv_b = jnp.broadcast_to(dinv[:, None], xs_ref.shape)
    xw = jnp.dot(x_ref[...], w_ref[...],
                 preferred_element_type=jnp.float32) + b_ref[...]
    dinvb_ref[...] = dinv_b
    xs_ref[...] = xw * dinv_b


def _bn_relu(agg, g_ref, be_ref, m_ref, v_ref):
    s = g_ref[...] * lax.rsqrt(v_ref[...] + EPS)
    t = be_ref[...] - m_ref[...] * s
    return jnp.maximum(agg * s + t, 0.0)


def _tc_mid_body(acc_ref, xs_ref, dinvb_ref, g_ref, be_ref, m_ref, v_ref,
                 w_ref, b_ref, out_ref):
    agg = (acc_ref[0] + acc_ref[1] + xs_ref[...]) * dinvb_ref[...]
    h = _bn_relu(agg, g_ref, be_ref, m_ref, v_ref)
    xw = jnp.dot(h, w_ref[...], preferred_element_type=jnp.float32) + b_ref[...]
    out_ref[...] = xw * dinvb_ref[...]


def _tc_final_body(acc_ref, xs_ref, dinvb_ref, g_ref, be_ref, m_ref, v_ref,
                   batch_ref, out_ref):
    agg = (acc_ref[0] + acc_ref[1] + xs_ref[...]) * dinvb_ref[...]
    h = _bn_relu(agg, g_ref, be_ref, m_ref, v_ref)
    gids = lax.broadcasted_iota(jnp.int32, (G, batch_ref.shape[1]),
                                0).astype(jnp.float32)
    p = jnp.where(batch_ref[...] == gids, 1.0, 0.0)
    sums = jnp.dot(p, h, preferred_element_type=jnp.float32)
    cnt = jnp.sum(p, axis=1, keepdims=True)
    out_ref[...] = sums / jnp.maximum(cnt, 1.0)


def _tc_call(body, out_shape):
    return pl.pallas_call(body, out_shape=out_shape)


def kernel(x, edge_index, batch,
           W1, b1, gamma1, beta1, mean1, var1,
           W2, b2, gamma2, beta2, mean2, var2,
           W3, b3, gamma3, beta3, mean3, var3):
    n, d = x.shape
    h = W1.shape[1]
    e = edge_index.shape[1]

    npad = _round_up(n + 1, TPC * WCHUNK)
    ep = _round_up(e, NTILES * NBUF * CHUNK)
    ept = ep // NTILES
    k = ept // CHUNK

    pad_e = ep - e
    pad_idx = n + jnp.arange(pad_e, dtype=jnp.int32) % (npad - n)
    src_p = jnp.concatenate([edge_index[0], pad_idx])
    dst_p = jnp.concatenate([edge_index[1], pad_idx])
    src3 = src_p.reshape(NTILES, k, CHUNK)
    dst3 = dst_p.reshape(NTILES, k, CHUNK)
    dstf = edge_index[1].reshape(NTILES, e // NTILES)
    x_p = jnp.zeros((npad, d), jnp.float32).at[:n].set(x)
    batch_row = jnp.concatenate(
        [batch, jnp.full((npad - n,), G, jnp.int32)]).astype(
            jnp.float32).reshape(1, npad)
    row = lambda v: v.reshape(1, h)

    mesh = plsc.VectorSubcoreMesh(core_axis_name="c", subcore_axis_name="s")

    deg_call = pl.kernel(
        _deg_body,
        out_type=jax.ShapeDtypeStruct((NTILES, npad), jnp.float32),
        mesh=mesh,
        compiler_params=pltpu.CompilerParams(needs_layout_passes=False),
        scratch_types=[
            pltpu.VMEM((e // NTILES,), jnp.int32),
            pltpu.VMEM((npad,), jnp.float32),
        ],
    )

    spmm_call = pl.kernel(
        _spmm_body,
        out_type=jax.ShapeDtypeStruct((2, npad, h), jnp.float32),
        mesh=mesh,
        scratch_types=[
            pltpu.VMEM((NSLOT, NBUF, CHUNK), jnp.int32),
            pltpu.VMEM((NSLOT, NBUF, CHUNK), jnp.int32),
            pltpu.VMEM((NBUF, CHUNK, h), jnp.float32),
            pltpu.VMEM_SHARED((npad, h), jnp.float32),
            pltpu.SemaphoreType.DMA((NBUF,)),
            pltpu.SemaphoreType.DMA((NSLOT,)),
        ],
    )

    deg_parts = deg_call(dstf)

    xs1, dinvb = _tc_call(
        _tc1_body,
        (jax.ShapeDtypeStruct((npad, h), jnp.float32),
         jax.ShapeDtypeStruct((npad, h), jnp.float32)),
    )(x_p, W1, row(b1), deg_parts)

    acc1 = spmm_call(src3, dst3, xs1)
    xs2 = _tc_call(_tc_mid_body, jax.ShapeDtypeStruct((npad, h), jnp.float32))(
        acc1, xs1, dinvb, row(gamma1), row(beta1), row(mean1), row(var1),
        W2, row(b2))

    acc2 = spmm_call(src3, dst3, xs2)
    xs3 = _tc_call(_tc_mid_body, jax.ShapeDtypeStruct((npad, h), jnp.float32))(
        acc2, xs2, dinvb, row(gamma2), row(beta2), row(mean2), row(var2),
        W3, row(b3))

    acc3 = spmm_call(src3, dst3, xs3)
    out = _tc_call(_tc_final_body, jax.ShapeDtypeStruct((G, h), jnp.float32))(
        acc3, xs3, dinvb, row(gamma3), row(beta3), row(mean3), row(var3),
        batch_row)
    return out


# speedup vs baseline: 1.1293x; 1.0104x over previous
import jax
import jax.numpy as jnp
from jax import lax
from jax.experimental import pallas as pl
from jax.experimental.pallas import tpu as pltpu
from jax.experimental.pallas import tpu_sc as plsc

EPS = 1e-5
G = 64
LANES = 16
CHUNK = 64
NBUF = 4
NSLOT = 4
NTILES = 32
TPC = 16
WCHUNK = 128


def _round_up(a, b):
    return (a + b - 1) // b * b


def _deg_body(dst_hbm, deg_hbm, dst_v, deg_v):
    cid = lax.axis_index("c")
    sid = lax.axis_index("s")
    wid = cid * TPC + sid
    npad = deg_v.shape[0]
    ept = dst_v.shape[0]

    def zero(i, carry):
        deg_v[pl.ds(i * LANES, LANES)] = jnp.zeros((LANES,), jnp.float32)
        return carry

    lax.fori_loop(0, npad // LANES, zero, 0)
    pltpu.sync_copy(dst_hbm.at[wid], dst_v)

    ones = jnp.ones((LANES,), jnp.float32)

    def accum(i, carry):
        idx = dst_v[pl.ds(i * LANES, LANES)]
        plsc.addupdate_scatter(deg_v, [idx], ones)
        return carry

    lax.fori_loop(0, ept // LANES, accum, 0)
    pltpu.sync_copy(deg_v, deg_hbm.at[wid])


def _spmm_body(src_hbm, dst_hbm, table_hbm, out_hbm, sidx_v, didx_v, rows_v,
               acc_sh, gsem, isem):
    cid = lax.axis_index("c")
    sid = lax.axis_index("s")
    wid = cid * TPC + sid
    k = src_hbm.shape[1]
    ng = k // NBUF
    npad = acc_sh.shape[0]
    rows_per_tile = npad // TPC

    def idx_issue(g, sl):
        gr = pl.ds(g * NBUF, NBUF)
        pltpu.async_copy(src_hbm.at[wid, gr], sidx_v.at[sl], isem.at[sl])
        pltpu.async_copy(dst_hbm.at[wid, gr], didx_v.at[sl], isem.at[sl])

    def idx_wait(sl):
        pltpu.make_async_copy(src_hbm.at[wid, pl.ds(0, NBUF)], sidx_v.at[sl],
                              isem.at[sl]).wait()
        pltpu.make_async_copy(dst_hbm.at[wid, pl.ds(0, NBUF)], didx_v.at[sl],
                              isem.at[sl]).wait()

    def zero(r, carry):
        for c in range(rows_v.shape[2] // LANES):
            rows_v[0, r, pl.ds(c * LANES, LANES)] = jnp.zeros((LANES,),
                                                              jnp.float32)
        return carry

    lax.fori_loop(0, CHUNK, zero, 0)
    base = sid * rows_per_tile
    for i in range(rows_per_tile // CHUNK):
        pltpu.sync_copy(rows_v.at[0], acc_sh.at[pl.ds(base + i * CHUNK,
                                                      CHUNK)])

    idx_issue(0, 0)
    idx_issue(1, 1)
    plsc.subcore_barrier()

    idx_wait(0)
    for b in range(NBUF):
        pltpu.async_copy(table_hbm.at[sidx_v.at[0, b]], rows_v.at[b],
                         gsem.at[b])

    def group(g, carry):
        csl = jnp.bitwise_and(g, NSLOT - 1)
        nsl = jnp.bitwise_and(g + 1, NSLOT - 1)

        @pl.when(g + 2 < ng)
        def _():
            idx_issue(g + 2, jnp.bitwise_and(g + 2, NSLOT - 1))

        @pl.when(g + 1 < ng)
        def _():
            idx_wait(nsl)

        for b in range(NBUF):
            pltpu.make_async_copy(table_hbm.at[sidx_v.at[csl, b]],
                                  rows_v.at[b], gsem.at[b]).wait()
            pltpu.sync_copy(rows_v.at[b], acc_sh.at[didx_v.at[csl, b]],
                            add=True)

            @pl.when(g + 1 < ng)
            def _():
                pltpu.async_copy(table_hbm.at[sidx_v.at[nsl, b]],
                                 rows_v.at[b], gsem.at[b])
        return carry

    lax.fori_loop(0, ng, group, 0)
    plsc.subcore_barrier()

    for i in range(rows_per_tile // WCHUNK):
        sl = pl.ds(base + i * WCHUNK, WCHUNK)
        pltpu.sync_copy(acc_sh.at[sl], out_hbm.at[cid, sl])


def _dinv_b(degp_ref, shape):
    deg = jnp.sum(degp_ref[...], axis=0) + 1.0
    dinv = lax.rsqrt(jnp.maximum(deg, 1.0))
    return jnp.broadcast_to(dinv[:, None], shape)


def _tc1_body(x_ref, w_ref, b_ref, degp_ref, xs_ref):
    xw = jnp.dot(x_ref[...], w_ref[...],
                 preferred_element_type=jnp.float32) + b_ref[...]
    xs_ref[...] = xw * _dinv_b(degp_ref, xw.shape)


def _bn_relu(agg, g_ref, be_ref, m_ref, v_ref):
    s = g_ref[...] * lax.rsqrt(v_ref[...] + EPS)
    t = be_ref[...] - m_ref[...] * s
    return jnp.maximum(agg * s + t, 0.0)


def _tc_mid_body(acc_ref, xs_ref, degp_ref, g_ref, be_ref, m_ref, v_ref,
                 w_ref, b_ref, out_ref):
    dinv_b = _dinv_b(degp_ref, xs_ref.shape)
    agg = (acc_ref[0] + acc_ref[1] + xs_ref[...]) * dinv_b
    h = _bn_relu(agg, g_ref, be_ref, m_ref, v_ref)
    xw = jnp.dot(h, w_ref[...], preferred_element_type=jnp.float32) + b_ref[...]
    out_ref[...] = xw * dinv_b


def _tc_final_body(acc_ref, xs_ref, degp_ref, g_ref, be_ref, m_ref, v_ref,
                   batch_ref, out_ref):
    agg = (acc_ref[0] + acc_ref[1] + xs_ref[...]) * _dinv_b(
        degp_ref, xs_ref.shape)
    h = _bn_relu(agg, g_ref, be_ref, m_ref, v_ref)
    gids = lax.broadcasted_iota(jnp.int32, (G, batch_ref.shape[1]),
                                0).astype(jnp.float32)
    p = jnp.where(batch_ref[...] == gids, 1.0, 0.0)
    sums = jnp.dot(p, h, preferred_element_type=jnp.float32)
    cnt = jnp.sum(p, axis=1, keepdims=True)
    out_ref[...] = sums / jnp.maximum(cnt, 1.0)


def _tc_call(body, out_shape):
    return pl.pallas_call(body, out_shape=out_shape)


def kernel(x, edge_index, batch,
           W1, b1, gamma1, beta1, mean1, var1,
           W2, b2, gamma2, beta2, mean2, var2,
           W3, b3, gamma3, beta3, mean3, var3):
    n, d = x.shape
    h = W1.shape[1]
    e = edge_index.shape[1]

    npad = _round_up(n + 1, TPC * WCHUNK)
    ep = _round_up(e, NTILES * NBUF * CHUNK)
    ept = ep // NTILES
    k = ept // CHUNK

    pad_e = ep - e
    pad_idx = n + jnp.arange(pad_e, dtype=jnp.int32) % (npad - n)
    src_p = jnp.concatenate([edge_index[0], pad_idx])
    dst_p = jnp.concatenate([edge_index[1], pad_idx])
    src3 = src_p.reshape(NTILES, k, CHUNK)
    dst3 = dst_p.reshape(NTILES, k, CHUNK)
    dstf = edge_index[1].reshape(NTILES, e // NTILES)
    x_p = jnp.zeros((npad, d), jnp.float32).at[:n].set(x)
    batch_row = jnp.concatenate(
        [batch, jnp.full((npad - n,), G, jnp.int32)]).astype(
            jnp.float32).reshape(1, npad)
    row = lambda v: v.reshape(1, h)

    mesh = plsc.VectorSubcoreMesh(core_axis_name="c", subcore_axis_name="s")

    deg_call = pl.kernel(
        _deg_body,
        out_type=jax.ShapeDtypeStruct((NTILES, npad), jnp.float32),
        mesh=mesh,
        compiler_params=pltpu.CompilerParams(needs_layout_passes=False),
        scratch_types=[
            pltpu.VMEM((e // NTILES,), jnp.int32),
            pltpu.VMEM((npad,), jnp.float32),
        ],
    )

    spmm_call = pl.kernel(
        _spmm_body,
        out_type=jax.ShapeDtypeStruct((2, npad, h), jnp.float32),
        mesh=mesh,
        scratch_types=[
            pltpu.VMEM((NSLOT, NBUF, CHUNK), jnp.int32),
            pltpu.VMEM((NSLOT, NBUF, CHUNK), jnp.int32),
            pltpu.VMEM((NBUF, CHUNK, h), jnp.float32),
            pltpu.VMEM_SHARED((npad, h), jnp.float32),
            pltpu.SemaphoreType.DMA((NBUF,)),
            pltpu.SemaphoreType.DMA((NSLOT,)),
        ],
    )

    deg_parts = deg_call(dstf)

    xs1 = _tc_call(_tc1_body, jax.ShapeDtypeStruct((npad, h), jnp.float32))(
        x_p, W1, row(b1), deg_parts)

    acc1 = spmm_call(src3, dst3, xs1)
    xs2 = _tc_call(_tc_mid_body, jax.ShapeDtypeStruct((npad, h), jnp.float32))(
        acc1, xs1, deg_parts, row(gamma1), row(beta1), row(mean1), row(var1),
        W2, row(b2))

    acc2 = spmm_call(src3, dst3, xs2)
    xs3 = _tc_call(_tc_mid_body, jax.ShapeDtypeStruct((npad, h), jnp.float32))(
        acc2, xs2, deg_parts, row(gamma2), row(beta2), row(mean2), row(var2),
        W3, row(b3))

    acc3 = spmm_call(src3, dst3, xs3)
    out = _tc_call(_tc_final_body, jax.ShapeDtypeStruct((G, h), jnp.float32))(
        acc3, xs3, deg_parts, row(gamma3), row(beta3), row(mean3), row(var3),
        batch_row)
    return out
